# baseline (speedup 1.0000x reference)
import functools

import jax
import jax.numpy as jnp
from jax import lax
from jax.experimental import pallas as pl
from jax.experimental.pallas import tpu as pltpu
from jax.experimental.pallas import tpu_sc as plsc

_N = 10000
_E = 320000
_D = 128
_R = 10
_L = 5
_B = 16

_NC = 2
_NS = 16
_GRP = 128
_OCT = 8
_GSZ = 1000
_NG = _N // _GSZ
_G_PER_SC = _NG // _NC
_LTRASH = _R * _GSZ
_TAB = _R * _GSZ + 16
_BUCKET_Q = _NS * _GRP * _OCT
_EPB = _E + _NG * _BUCKET_Q
_EPADC = 327680
_SPREAD = 64
_ZB = 64

_WSH = _N // _NS // 8 * 8
_WSH_LAST = _N - (_NS - 1) * _WSH
_ZSH = _WSH
_ZSH_LAST = _TAB - (_NS - 1) * _ZSH


def _sc_mesh():
    return plsc.VectorSubcoreMesh(core_axis_name="c", subcore_axis_name="s")


def _zero_table(zbuf, table, base, nrows):
    po = 0
    while po < nrows:
        sz = min(_ZB, nrows - po)
        pltpu.sync_copy(zbuf.at[pl.ds(0, sz)], table.at[pl.ds(base + po, sz)])
        po += sz


def _write_back(table, rows, out_at, base, nrows):
    po = 0
    while po < nrows:
        sz = min(_GRP, nrows - po)
        pltpu.sync_copy(table.at[pl.ds(base + po, sz)], rows.at[pl.ds(0, sz)])
        pltpu.sync_copy(rows.at[pl.ds(0, sz)], out_at(base + po, sz))
        po += sz


def _stream_octets(noct, erow0, idx_src, idx_seg, gsrc, table,
                   src8, seg8, rows_a, rows_b, sem_a, sem_b):
    bufs = (rows_a, rows_b)
    sems = (sem_a, sem_b)

    def octet(i, _):
        erow = pl.multiple_of(erow0 + i * _OCT, _OCT)
        pltpu.sync_copy(idx_src.at[pl.ds(erow, _OCT), :], src8)
        pltpu.sync_copy(idx_seg.at[pl.ds(erow, _OCT), :], seg8)
        d = [None, None]
        d[0] = pltpu.async_copy(gsrc.at[src8.at[0]], bufs[0], sems[0])
        for g in range(_OCT):
            cur = g % 2
            if g < _OCT - 1:
                nxt = (g + 1) % 2
                d[nxt] = pltpu.async_copy(
                    gsrc.at[src8.at[g + 1]], bufs[nxt], sems[nxt])
            d[cur].wait()
            pltpu.sync_copy(bufs[cur], table.at[seg8.at[g]], add=True)
        return _

    lax.fori_loop(0, noct, octet, None)


def _agg_body(h, srcb, segb, pbrv, trv, zsrc, agg,
              table, zbuf, pbuf, tbuf, src8, seg8, rows_a, rows_b,
              sem_a, sem_b):
    cid = lax.axis_index("c")
    sid = lax.axis_index("s")
    is_last = sid == _NS - 1
    pltpu.sync_copy(zsrc, zbuf)
    pltpu.sync_copy(pbrv, pbuf)
    pltpu.sync_copy(trv, tbuf)

    lanes = lax.iota(jnp.int32, 16)
    pbvec = pbuf[...]
    trvec = tbuf[...]

    for j in range(_G_PER_SC):
        k = cid * _G_PER_SC + j
        pbr_k = jnp.sum(jnp.where(lanes == k, pbvec, 0))
        tr_k = jnp.sum(jnp.where(lanes == k, trvec, 0))

        pl.when(jnp.logical_not(is_last))(
            lambda: _zero_table(zbuf, table, sid * _ZSH, _ZSH))
        pl.when(is_last)(
            lambda: _zero_table(zbuf, table, sid * _ZSH, _ZSH_LAST))
        plsc.subcore_barrier()

        erow0 = pbr_k + sid * tr_k * _OCT
        _stream_octets(tr_k, erow0, srcb, segb, h, table,
                       src8, seg8, rows_a, rows_b, sem_a, sem_b)
        plsc.subcore_barrier()

        out_at = lambda off, sz: agg.at[k, pl.ds(off, sz), :]
        pl.when(jnp.logical_not(is_last))(
            lambda: _write_back(table, rows_a, out_at, sid * _WSH, _WSH))
        pl.when(is_last)(
            lambda: _write_back(table, rows_a, out_at, sid * _WSH, _WSH_LAST))
        plsc.subcore_barrier()


def _make_agg_call():
    return pl.kernel(
        _agg_body,
        out_type=jax.ShapeDtypeStruct((_NG, _R * _GSZ, _D), jnp.float32),
        mesh=_sc_mesh(),
        compiler_params=pltpu.CompilerParams(needs_layout_passes=False),
        scratch_types=[
            pltpu.VMEM_SHARED((_TAB, _D), jnp.float32),
            pltpu.VMEM((_ZB, _D), jnp.float32),
            pltpu.VMEM((16,), jnp.int32),
            pltpu.VMEM((16,), jnp.int32),
            pltpu.VMEM((_OCT, _GRP), jnp.int32),
            pltpu.VMEM((_OCT, _GRP), jnp.int32),
            pltpu.VMEM((_GRP, _D), jnp.float32),
            pltpu.VMEM((_GRP, _D), jnp.float32),
            pltpu.SemaphoreType.DMA,
            pltpu.SemaphoreType.DMA,
        ],
    )


_CNT_OCTETS = _EPADC // (_NC * _NS) // _GRP // _OCT


def _cnt_body(eye, etp, dstp, zsrc, cnt,
              table, zbuf, et8, dst8, rows_a, rows_b, sem_a, sem_b):
    cid = lax.axis_index("c")
    sid = lax.axis_index("s")
    is_last = sid == _NS - 1
    pltpu.sync_copy(zsrc, zbuf)

    pl.when(jnp.logical_not(is_last))(
        lambda: _zero_table(zbuf, table, sid * _ZSH, _ZSH))
    pl.when(is_last)(
        lambda: _zero_table(zbuf, table, sid * _ZSH, _ZSH_LAST))
    plsc.subcore_barrier()

    erow0 = (cid * _NS + sid) * (_CNT_OCTETS * _OCT)
    _stream_octets(_CNT_OCTETS, erow0, etp, dstp, eye, table,
                   et8, dst8, rows_a, rows_b, sem_a, sem_b)
    plsc.subcore_barrier()

    out_at = lambda off, sz: cnt.at[cid, pl.ds(off, sz), :]
    pl.when(jnp.logical_not(is_last))(
        lambda: _write_back(table, rows_a, out_at, sid * _WSH, _WSH))
    pl.when(is_last)(
        lambda: _write_back(table, rows_a, out_at, sid * _WSH, _WSH_LAST))


def _make_cnt_call():
    return pl.kernel(
        _cnt_body,
        out_type=jax.ShapeDtypeStruct((_NC, _N, _D), jnp.float32),
        mesh=_sc_mesh(),
        compiler_params=pltpu.CompilerParams(needs_layout_passes=False),
        scratch_types=[
            pltpu.VMEM_SHARED((_TAB, _D), jnp.float32),
            pltpu.VMEM((_ZB, _D), jnp.float32),
            pltpu.VMEM((_OCT, _GRP), jnp.int32),
            pltpu.VMEM((_OCT, _GRP), jnp.int32),
            pltpu.VMEM((_GRP, _D), jnp.float32),
            pltpu.VMEM((_GRP, _D), jnp.float32),
            pltpu.SemaphoreType.DMA,
            pltpu.SemaphoreType.DMA,
        ],
    )


def _layer_body(relu, h_ref, agg_ref, cnt_ref, wr_ref, wrel_ref, b_ref, o_ref):
    cnt = cnt_ref[0] + cnt_ref[1]
    icnt = 1.0 / jnp.maximum(cnt, 1.0)
    acc = jnp.dot(h_ref[...], wr_ref[...], preferred_element_type=jnp.float32)
    acc = acc + b_ref[...]
    a = agg_ref[0]
    for r in range(_R):
        ar = a[r * _GSZ:(r + 1) * _GSZ, :] * icnt[:, r:r + 1]
        acc = acc + jnp.dot(ar, wrel_ref[r], preferred_element_type=jnp.float32)
    if relu:
        acc = jnp.maximum(acc, 0.0)
    o_ref[...] = acc


def _make_layer_call(relu):
    return pl.pallas_call(
        functools.partial(_layer_body, relu),
        grid=(_NG,),
        in_specs=[
            pl.BlockSpec((_GSZ, _D), lambda i: (i, 0)),
            pl.BlockSpec((1, _R * _GSZ, _D), lambda i: (i, 0, 0)),
            pl.BlockSpec((_NC, _GSZ, _D), lambda i: (0, i, 0)),
            pl.BlockSpec((_D, _D), lambda i: (0, 0)),
            pl.BlockSpec((_R, _D, _D), lambda i: (0, 0, 0)),
            pl.BlockSpec((1, _D), lambda i: (0, 0)),
        ],
        out_specs=pl.BlockSpec((_GSZ, _D), lambda i: (i, 0)),
        out_shape=jax.ShapeDtypeStruct((_N, _D), jnp.float32),
    )


def _extract_body(h_ref, b_ref, i_ref, o_ref):
    bt = b_ref[...]
    i0 = i_ref[0]
    i1 = i_ref[1]
    for b in range(_B):
        offs_b = jnp.sum((bt < b).astype(jnp.int32))
        r0 = jnp.clip(offs_b + i0[b], 0, _N - 1)
        r1 = jnp.clip(offs_b + i1[b], 0, _N - 1)
        o_ref[b:b + 1, 0:_D] = h_ref[pl.ds(r0, 1), :]
        o_ref[b:b + 1, _D:2 * _D] = h_ref[pl.ds(r1, 1), :]


def _make_extract_call():
    return pl.pallas_call(
        _extract_body,
        out_shape=jax.ShapeDtypeStruct((_B, 2 * _D), jnp.float32),
    )


def kernel(x, edge_index, edge_type, batch, inds, Wrel, Wroot, bias):
    src = edge_index[0].astype(jnp.int32)
    dst = edge_index[1].astype(jnp.int32)
    et = edge_type.reshape(-1).astype(jnp.int32)

    g = dst // _GSZ
    segl = et * _GSZ + (dst - g * _GSZ)
    v = ((g.astype(jnp.uint32) << 28)
         | (src.astype(jnp.uint32) << 14)
         | segl.astype(jnp.uint32))
    vs = jnp.sort(v)
    bounds = jnp.searchsorted(
        vs, (jnp.arange(_NG + 1, dtype=jnp.uint32) << 28)).astype(jnp.int32)
    cnt_g = bounds[1:] - bounds[:-1]
    pc = (cnt_g + _BUCKET_Q - 1) // _BUCKET_Q * _BUCKET_Q
    pb = jnp.concatenate(
        [jnp.zeros((1,), jnp.int32), jnp.cumsum(pc)[:-1].astype(jnp.int32)])
    q = jnp.arange(_EPB, dtype=jnp.int32)
    k_of_q = jnp.sum((q[:, None] >= pb[None, :]).astype(jnp.int32), axis=1) - 1
    t = q - pb[k_of_q]
    idx = jnp.where(t < cnt_g[k_of_q], bounds[:-1][k_of_q] + t, _E)
    vse = jnp.concatenate([vs, jnp.full((1,), _LTRASH, jnp.uint32)])
    vb = vse[idx]
    srcb = ((vb >> 14) & 0x3FFF).astype(jnp.int32).reshape(-1, _GRP)
    segb = (vb & 0x3FFF).astype(jnp.int32).reshape(-1, _GRP)
    pbrv = jnp.zeros((16,), jnp.int32).at[:_NG].set(pb // _GRP)
    trv = jnp.zeros((16,), jnp.int32).at[:_NG].set(pc // _BUCKET_Q)

    padc = _EPADC - _E
    etp = jnp.concatenate([et, jnp.zeros((padc,), jnp.int32)])
    etp = (etp + 16 * (jnp.arange(_EPADC, dtype=jnp.int32) % _SPREAD)
           ).reshape(-1, _GRP)
    dstp = jnp.concatenate(
        [dst, jnp.full((padc,), _N, jnp.int32)]).reshape(-1, _GRP)
    eye1 = jnp.zeros((16, _D), jnp.float32).at[
        jnp.arange(16), jnp.arange(16)].set(1.0)
    eye = jnp.tile(eye1, (_SPREAD, 1))
    zsrc = jnp.zeros((_ZB, _D), jnp.float32)
    bias2 = bias.reshape(_L, 1, _D)

    cnt = _make_cnt_call()(eye, etp, dstp, zsrc)

    agg_call = _make_agg_call()
    h = x
    for l in range(_L):
        agg = agg_call(h, srcb, segb, pbrv, trv, zsrc)
        h = _make_layer_call(l < _L - 1)(
            h, agg, cnt, Wroot[l], Wrel[l], bias2[l])

    return _make_extract_call()(h, batch.reshape(_N // 16, 16), inds)


# speedup vs baseline: 1.2453x; 1.2453x over previous
import functools

import jax
import jax.numpy as jnp
from jax import lax
from jax.experimental import pallas as pl
from jax.experimental.pallas import tpu as pltpu
from jax.experimental.pallas import tpu_sc as plsc

_N = 10000
_E = 320000
_D = 128
_R = 10
_L = 5
_B = 16

_NC = 2
_NS = 16
_GRP = 128
_OCT = 8
_GSZ = 1000
_NG = _N // _GSZ
_G_PER_SC = _NG // _NC
_LTRASH = _R * _GSZ
_TAB = _R * _GSZ + 16
_BUCKET_Q = _NS * _GRP * _OCT
_EPB = _E + _NG * _BUCKET_Q
_EPADC = 327680
_SPREAD = 64
_ZB = 64

_WSH = _N // _NS // 8 * 8
_WSH_LAST = _N - (_NS - 1) * _WSH
_ZSH = _WSH
_ZSH_LAST = _TAB - (_NS - 1) * _ZSH


def _sc_mesh():
    return plsc.VectorSubcoreMesh(core_axis_name="c", subcore_axis_name="s")


def _zero_table(zbuf, table, base, nrows):
    po = 0
    while po < nrows:
        sz = min(_ZB, nrows - po)
        pltpu.sync_copy(zbuf.at[pl.ds(0, sz)], table.at[pl.ds(base + po, sz)])
        po += sz


def _write_back(table, rows, out_at, base, nrows):
    po = 0
    while po < nrows:
        sz = min(_GRP, nrows - po)
        pltpu.sync_copy(table.at[pl.ds(base + po, sz)], rows.at[pl.ds(0, sz)])
        pltpu.sync_copy(rows.at[pl.ds(0, sz)], out_at(base + po, sz))
        po += sz


def _stream_octets(noct, erow0, idx_src, idx_seg, gsrc, table,
                   src8, seg8, rows_a, rows_b, sem_a, sem_b):
    bufs = (rows_a, rows_b)
    sems = (sem_a, sem_b)

    def octet(i, _):
        erow = pl.multiple_of(erow0 + i * _OCT, _OCT)
        pltpu.sync_copy(idx_src.at[pl.ds(erow, _OCT), :], src8)
        pltpu.sync_copy(idx_seg.at[pl.ds(erow, _OCT), :], seg8)
        d = [None, None]
        d[0] = pltpu.async_copy(gsrc.at[src8.at[0]], bufs[0], sems[0])
        for g in range(_OCT):
            cur = g % 2
            if g < _OCT - 1:
                nxt = (g + 1) % 2
                d[nxt] = pltpu.async_copy(
                    gsrc.at[src8.at[g + 1]], bufs[nxt], sems[nxt])
            d[cur].wait()
            pltpu.sync_copy(bufs[cur], table.at[seg8.at[g]], add=True)
        return _

    lax.fori_loop(0, noct, octet, None)


def _agg_body(h, srcb, segb, pbrv, trv, zsrc, agg,
              table, zbuf, pbuf, tbuf, src8, seg8, rows_a, rows_b,
              sem_a, sem_b):
    cid = lax.axis_index("c")
    sid = lax.axis_index("s")
    is_last = sid == _NS - 1
    pltpu.sync_copy(zsrc, zbuf)
    pltpu.sync_copy(pbrv, pbuf)
    pltpu.sync_copy(trv, tbuf)

    lanes = lax.iota(jnp.int32, 16)
    pbvec = pbuf[...]
    trvec = tbuf[...]

    for j in range(_G_PER_SC):
        k = cid * _G_PER_SC + j
        pbr_k = jnp.sum(jnp.where(lanes == k, pbvec, 0))
        tr_k = jnp.sum(jnp.where(lanes == k, trvec, 0))

        pl.when(jnp.logical_not(is_last))(
            lambda: _zero_table(zbuf, table, sid * _ZSH, _ZSH))
        pl.when(is_last)(
            lambda: _zero_table(zbuf, table, sid * _ZSH, _ZSH_LAST))
        plsc.subcore_barrier()

        erow0 = pbr_k + sid * tr_k * _OCT
        _stream_octets(tr_k, erow0, srcb, segb, h, table,
                       src8, seg8, rows_a, rows_b, sem_a, sem_b)
        plsc.subcore_barrier()

        out_at = lambda off, sz: agg.at[k, pl.ds(off, sz), :]
        pl.when(jnp.logical_not(is_last))(
            lambda: _write_back(table, rows_a, out_at, sid * _WSH, _WSH))
        pl.when(is_last)(
            lambda: _write_back(table, rows_a, out_at, sid * _WSH, _WSH_LAST))
        plsc.subcore_barrier()


def _make_agg_call():
    return pl.kernel(
        _agg_body,
        out_type=jax.ShapeDtypeStruct((_NG, _R * _GSZ, _D), jnp.float32),
        mesh=_sc_mesh(),
        compiler_params=pltpu.CompilerParams(needs_layout_passes=False),
        scratch_types=[
            pltpu.VMEM_SHARED((_TAB, _D), jnp.float32),
            pltpu.VMEM((_ZB, _D), jnp.float32),
            pltpu.VMEM((16,), jnp.int32),
            pltpu.VMEM((16,), jnp.int32),
            pltpu.VMEM((_OCT, _GRP), jnp.int32),
            pltpu.VMEM((_OCT, _GRP), jnp.int32),
            pltpu.VMEM((_GRP, _D), jnp.float32),
            pltpu.VMEM((_GRP, _D), jnp.float32),
            pltpu.SemaphoreType.DMA,
            pltpu.SemaphoreType.DMA,
        ],
    )


_CNT_OCTETS = _EPADC // (_NC * _NS) // _GRP // _OCT


def _cnt_body(eye, etp, dstp, zsrc, cnt,
              table, zbuf, et8, dst8, rows_a, rows_b, sem_a, sem_b):
    cid = lax.axis_index("c")
    sid = lax.axis_index("s")
    is_last = sid == _NS - 1
    pltpu.sync_copy(zsrc, zbuf)

    pl.when(jnp.logical_not(is_last))(
        lambda: _zero_table(zbuf, table, sid * _ZSH, _ZSH))
    pl.when(is_last)(
        lambda: _zero_table(zbuf, table, sid * _ZSH, _ZSH_LAST))
    plsc.subcore_barrier()

    erow0 = (cid * _NS + sid) * (_CNT_OCTETS * _OCT)
    _stream_octets(_CNT_OCTETS, erow0, etp, dstp, eye, table,
                   et8, dst8, rows_a, rows_b, sem_a, sem_b)
    plsc.subcore_barrier()

    out_at = lambda off, sz: cnt.at[cid, pl.ds(off, sz), :]
    pl.when(jnp.logical_not(is_last))(
        lambda: _write_back(table, rows_a, out_at, sid * _WSH, _WSH))
    pl.when(is_last)(
        lambda: _write_back(table, rows_a, out_at, sid * _WSH, _WSH_LAST))


def _make_cnt_call():
    return pl.kernel(
        _cnt_body,
        out_type=jax.ShapeDtypeStruct((_NC, _N, _D), jnp.float32),
        mesh=_sc_mesh(),
        compiler_params=pltpu.CompilerParams(needs_layout_passes=False),
        scratch_types=[
            pltpu.VMEM_SHARED((_TAB, _D), jnp.float32),
            pltpu.VMEM((_ZB, _D), jnp.float32),
            pltpu.VMEM((_OCT, _GRP), jnp.int32),
            pltpu.VMEM((_OCT, _GRP), jnp.int32),
            pltpu.VMEM((_GRP, _D), jnp.float32),
            pltpu.VMEM((_GRP, _D), jnp.float32),
            pltpu.SemaphoreType.DMA,
            pltpu.SemaphoreType.DMA,
        ],
    )


def _layer_body(relu, h_ref, agg_ref, cnt_ref, wr_ref, wrel_ref, b_ref, o_ref):
    cnt = cnt_ref[0] + cnt_ref[1]
    icnt = 1.0 / jnp.maximum(cnt, 1.0)
    acc = jnp.dot(h_ref[...], wr_ref[...], preferred_element_type=jnp.float32)
    acc = acc + b_ref[...]
    a = agg_ref[0]
    for r in range(_R):
        ar = a[r * _GSZ:(r + 1) * _GSZ, :] * icnt[:, r:r + 1]
        acc = acc + jnp.dot(ar, wrel_ref[r], preferred_element_type=jnp.float32)
    if relu:
        acc = jnp.maximum(acc, 0.0)
    o_ref[...] = acc


def _make_layer_call(relu):
    return pl.pallas_call(
        functools.partial(_layer_body, relu),
        grid=(_NG,),
        in_specs=[
            pl.BlockSpec((_GSZ, _D), lambda i: (i, 0)),
            pl.BlockSpec((1, _R * _GSZ, _D), lambda i: (i, 0, 0)),
            pl.BlockSpec((_NC, _GSZ, _D), lambda i: (0, i, 0)),
            pl.BlockSpec((_D, _D), lambda i: (0, 0)),
            pl.BlockSpec((_R, _D, _D), lambda i: (0, 0, 0)),
            pl.BlockSpec((1, _D), lambda i: (0, 0)),
        ],
        out_specs=pl.BlockSpec((_GSZ, _D), lambda i: (i, 0)),
        out_shape=jax.ShapeDtypeStruct((_N, _D), jnp.float32),
    )


def _extract_body(h_ref, b_ref, i_ref, o_ref):
    bt = b_ref[...]
    i0 = i_ref[0]
    i1 = i_ref[1]
    for b in range(_B):
        offs_b = jnp.sum((bt < b).astype(jnp.int32))
        r0 = jnp.clip(offs_b + i0[b], 0, _N - 1)
        r1 = jnp.clip(offs_b + i1[b], 0, _N - 1)
        o_ref[b:b + 1, 0:_D] = h_ref[pl.ds(r0, 1), :]
        o_ref[b:b + 1, _D:2 * _D] = h_ref[pl.ds(r1, 1), :]


def _make_extract_call():
    return pl.pallas_call(
        _extract_body,
        out_shape=jax.ShapeDtypeStruct((_B, 2 * _D), jnp.float32),
    )


def kernel(x, edge_index, edge_type, batch, inds, Wrel, Wroot, bias):
    src = edge_index[0].astype(jnp.int32)
    dst = edge_index[1].astype(jnp.int32)
    et = edge_type.reshape(-1).astype(jnp.int32)

    g = dst // _GSZ
    segl = et * _GSZ + (dst - g * _GSZ)
    oh = g[:, None] == jnp.arange(_NG, dtype=jnp.int32)[None, :]
    cnt_g = jnp.sum(oh.astype(jnp.int32), axis=0)
    pc = (cnt_g + _BUCKET_Q - 1) // _BUCKET_Q * _BUCKET_Q
    pb = jnp.concatenate(
        [jnp.zeros((1,), jnp.int32), jnp.cumsum(pc)[:-1].astype(jnp.int32)])
    pad_need_cum = jnp.cumsum(pc - cnt_g)
    padt = _EPB - _E
    padkeys = jnp.searchsorted(
        pad_need_cum, jnp.arange(padt, dtype=jnp.int32), side="right"
    ).astype(jnp.int32)
    keys = jnp.concatenate([g, padkeys])
    vals = jnp.concatenate([(src << 14) | segl,
                            jnp.full((padt,), _LTRASH, jnp.int32)])
    _, vb = lax.sort((keys, vals), num_keys=1, is_stable=True)
    srcb = (vb >> 14).reshape(-1, _GRP)
    segb = (vb & 0x3FFF).reshape(-1, _GRP)
    pbrv = jnp.zeros((16,), jnp.int32).at[:_NG].set(pb // _GRP)
    trv = jnp.zeros((16,), jnp.int32).at[:_NG].set(pc // _BUCKET_Q)

    padc = _EPADC - _E
    etp = jnp.concatenate([et, jnp.zeros((padc,), jnp.int32)])
    etp = (etp + 16 * (jnp.arange(_EPADC, dtype=jnp.int32) % _SPREAD)
           ).reshape(-1, _GRP)
    dstp = jnp.concatenate(
        [dst, jnp.full((padc,), _N, jnp.int32)]).reshape(-1, _GRP)
    eye1 = jnp.zeros((16, _D), jnp.float32).at[
        jnp.arange(16), jnp.arange(16)].set(1.0)
    eye = jnp.tile(eye1, (_SPREAD, 1))
    zsrc = jnp.zeros((_ZB, _D), jnp.float32)
    bias2 = bias.reshape(_L, 1, _D)

    cnt = _make_cnt_call()(eye, etp, dstp, zsrc)

    agg_call = _make_agg_call()
    h = x
    for l in range(_L):
        agg = agg_call(h, srcb, segb, pbrv, trv, zsrc)
        h = _make_layer_call(l < _L - 1)(
            h, agg, cnt, Wroot[l], Wrel[l], bias2[l])

    return _make_extract_call()(h, batch.reshape(_N // 16, 16), inds)


# speedup vs baseline: 1.2458x; 1.0004x over previous
import functools

import jax
import jax.numpy as jnp
from jax import lax
from jax.experimental import pallas as pl
from jax.experimental.pallas import tpu as pltpu
from jax.experimental.pallas import tpu_sc as plsc

_N = 10000
_E = 320000
_D = 128
_R = 10
_L = 5
_B = 16

_NC = 2
_NS = 16
_GRP = 128
_OCT = 8
_GSZ = 1000
_NG = _N // _GSZ
_G_PER_SC = _NG // _NC
_LTRASH = _R * _GSZ
_TAB = _R * _GSZ + 16
_BUCKET_Q = _NS * _GRP * _OCT
_EPB = _E + _NG * _BUCKET_Q
_EPADC = 327680
_SPREAD = 64
_ZB = 64

_WSH = _N // _NS // 8 * 8
_WSH_LAST = _N - (_NS - 1) * _WSH
_ZSH = _WSH
_ZSH_LAST = _TAB - (_NS - 1) * _ZSH


def _sc_mesh():
    return plsc.VectorSubcoreMesh(core_axis_name="c", subcore_axis_name="s")


def _zero_table(zbuf, table, base, nrows):
    po = 0
    while po < nrows:
        sz = min(_ZB, nrows - po)
        pltpu.sync_copy(zbuf.at[pl.ds(0, sz)], table.at[pl.ds(base + po, sz)])
        po += sz


def _write_back(table, rows, out_at, base, nrows):
    po = 0
    while po < nrows:
        sz = min(_GRP, nrows - po)
        pltpu.sync_copy(table.at[pl.ds(base + po, sz)], rows.at[pl.ds(0, sz)])
        pltpu.sync_copy(rows.at[pl.ds(0, sz)], out_at(base + po, sz))
        po += sz


def _stream_octets(noct, erow0, idx_src, idx_seg, gsrc, table,
                   src8, seg8, rows_a, rows_b, sem_a, sem_b, ssem_a, ssem_b):
    bufs = (rows_a, rows_b)
    sems = (sem_a, sem_b)
    ssems = (ssem_a, ssem_b)

    def octet(i, _):
        erow = pl.multiple_of(erow0 + i * _OCT, _OCT)
        pltpu.sync_copy(idx_src.at[pl.ds(erow, _OCT), :], src8)
        pltpu.sync_copy(idx_seg.at[pl.ds(erow, _OCT), :], seg8)
        gd = [None] * _OCT
        sd = [None] * _OCT
        gd[0] = pltpu.async_copy(gsrc.at[src8.at[0]], bufs[0], sems[0])
        for g in range(_OCT):
            cur = g % 2
            nxt = (g + 1) % 2
            if g < _OCT - 1:
                if g >= 1:
                    sd[g - 1].wait()
                gd[g + 1] = pltpu.async_copy(
                    gsrc.at[src8.at[g + 1]], bufs[nxt], sems[nxt])
            gd[g].wait()
            sd[g] = pltpu.async_copy(
                bufs[cur], table.at[seg8.at[g]], ssems[cur], add=True)
        sd[_OCT - 2].wait()
        sd[_OCT - 1].wait()
        return _

    lax.fori_loop(0, noct, octet, None)


def _agg_body(h, srcb, segb, pbrv, trv, zsrc, agg,
              table, zbuf, pbuf, tbuf, src8, seg8, rows_a, rows_b,
              sem_a, sem_b, ssem_a, ssem_b):
    cid = lax.axis_index("c")
    sid = lax.axis_index("s")
    is_last = sid == _NS - 1
    pltpu.sync_copy(zsrc, zbuf)
    pltpu.sync_copy(pbrv, pbuf)
    pltpu.sync_copy(trv, tbuf)

    lanes = lax.iota(jnp.int32, 16)
    pbvec = pbuf[...]
    trvec = tbuf[...]

    for j in range(_G_PER_SC):
        k = cid * _G_PER_SC + j
        pbr_k = jnp.sum(jnp.where(lanes == k, pbvec, 0))
        tr_k = jnp.sum(jnp.where(lanes == k, trvec, 0))

        pl.when(jnp.logical_not(is_last))(
            lambda: _zero_table(zbuf, table, sid * _ZSH, _ZSH))
        pl.when(is_last)(
            lambda: _zero_table(zbuf, table, sid * _ZSH, _ZSH_LAST))
        plsc.subcore_barrier()

        erow0 = pbr_k + sid * tr_k * _OCT
        _stream_octets(tr_k, erow0, srcb, segb, h, table,
                       src8, seg8, rows_a, rows_b, sem_a, sem_b,
                       ssem_a, ssem_b)
        plsc.subcore_barrier()

        out_at = lambda off, sz: agg.at[k, pl.ds(off, sz), :]
        pl.when(jnp.logical_not(is_last))(
            lambda: _write_back(table, rows_a, out_at, sid * _WSH, _WSH))
        pl.when(is_last)(
            lambda: _write_back(table, rows_a, out_at, sid * _WSH, _WSH_LAST))
        plsc.subcore_barrier()


def _make_agg_call():
    return pl.kernel(
        _agg_body,
        out_type=jax.ShapeDtypeStruct((_NG, _R * _GSZ, _D), jnp.float32),
        mesh=_sc_mesh(),
        compiler_params=pltpu.CompilerParams(needs_layout_passes=False),
        scratch_types=[
            pltpu.VMEM_SHARED((_TAB, _D), jnp.float32),
            pltpu.VMEM((_ZB, _D), jnp.float32),
            pltpu.VMEM((16,), jnp.int32),
            pltpu.VMEM((16,), jnp.int32),
            pltpu.VMEM((_OCT, _GRP), jnp.int32),
            pltpu.VMEM((_OCT, _GRP), jnp.int32),
            pltpu.VMEM((_GRP, _D), jnp.float32),
            pltpu.VMEM((_GRP, _D), jnp.float32),
            pltpu.SemaphoreType.DMA,
            pltpu.SemaphoreType.DMA,
            pltpu.SemaphoreType.DMA,
            pltpu.SemaphoreType.DMA,
        ],
    )


_CNT_OCTETS = _EPADC // (_NC * _NS) // _GRP // _OCT


def _cnt_body(eye, etp, dstp, zsrc, cnt,
              table, zbuf, et8, dst8, rows_a, rows_b, sem_a, sem_b,
              ssem_a, ssem_b):
    cid = lax.axis_index("c")
    sid = lax.axis_index("s")
    is_last = sid == _NS - 1
    pltpu.sync_copy(zsrc, zbuf)

    pl.when(jnp.logical_not(is_last))(
        lambda: _zero_table(zbuf, table, sid * _ZSH, _ZSH))
    pl.when(is_last)(
        lambda: _zero_table(zbuf, table, sid * _ZSH, _ZSH_LAST))
    plsc.subcore_barrier()

    erow0 = (cid * _NS + sid) * (_CNT_OCTETS * _OCT)
    _stream_octets(_CNT_OCTETS, erow0, etp, dstp, eye, table,
                   et8, dst8, rows_a, rows_b, sem_a, sem_b, ssem_a, ssem_b)
    plsc.subcore_barrier()

    out_at = lambda off, sz: cnt.at[cid, pl.ds(off, sz), :]
    pl.when(jnp.logical_not(is_last))(
        lambda: _write_back(table, rows_a, out_at, sid * _WSH, _WSH))
    pl.when(is_last)(
        lambda: _write_back(table, rows_a, out_at, sid * _WSH, _WSH_LAST))


def _make_cnt_call():
    return pl.kernel(
        _cnt_body,
        out_type=jax.ShapeDtypeStruct((_NC, _N, _D), jnp.float32),
        mesh=_sc_mesh(),
        compiler_params=pltpu.CompilerParams(needs_layout_passes=False),
        scratch_types=[
            pltpu.VMEM_SHARED((_TAB, _D), jnp.float32),
            pltpu.VMEM((_ZB, _D), jnp.float32),
            pltpu.VMEM((_OCT, _GRP), jnp.int32),
            pltpu.VMEM((_OCT, _GRP), jnp.int32),
            pltpu.VMEM((_GRP, _D), jnp.float32),
            pltpu.VMEM((_GRP, _D), jnp.float32),
            pltpu.SemaphoreType.DMA,
            pltpu.SemaphoreType.DMA,
            pltpu.SemaphoreType.DMA,
            pltpu.SemaphoreType.DMA,
        ],
    )


def _layer_body(relu, h_ref, agg_ref, cnt_ref, wr_ref, wrel_ref, b_ref, o_ref):
    cnt = cnt_ref[0] + cnt_ref[1]
    icnt = 1.0 / jnp.maximum(cnt, 1.0)
    acc = jnp.dot(h_ref[...], wr_ref[...], preferred_element_type=jnp.float32)
    acc = acc + b_ref[...]
    a = agg_ref[0]
    for r in range(_R):
        ar = a[r * _GSZ:(r + 1) * _GSZ, :] * icnt[:, r:r + 1]
        acc = acc + jnp.dot(ar, wrel_ref[r], preferred_element_type=jnp.float32)
    if relu:
        acc = jnp.maximum(acc, 0.0)
    o_ref[...] = acc


def _make_layer_call(relu):
    return pl.pallas_call(
        functools.partial(_layer_body, relu),
        grid=(_NG,),
        in_specs=[
            pl.BlockSpec((_GSZ, _D), lambda i: (i, 0)),
            pl.BlockSpec((1, _R * _GSZ, _D), lambda i: (i, 0, 0)),
            pl.BlockSpec((_NC, _GSZ, _D), lambda i: (0, i, 0)),
            pl.BlockSpec((_D, _D), lambda i: (0, 0)),
            pl.BlockSpec((_R, _D, _D), lambda i: (0, 0, 0)),
            pl.BlockSpec((1, _D), lambda i: (0, 0)),
        ],
        out_specs=pl.BlockSpec((_GSZ, _D), lambda i: (i, 0)),
        out_shape=jax.ShapeDtypeStruct((_N, _D), jnp.float32),
    )


def _extract_body(h_ref, b_ref, i_ref, o_ref):
    bt = b_ref[...]
    i0 = i_ref[0]
    i1 = i_ref[1]
    for b in range(_B):
        offs_b = jnp.sum((bt < b).astype(jnp.int32))
        r0 = jnp.clip(offs_b + i0[b], 0, _N - 1)
        r1 = jnp.clip(offs_b + i1[b], 0, _N - 1)
        o_ref[b:b + 1, 0:_D] = h_ref[pl.ds(r0, 1), :]
        o_ref[b:b + 1, _D:2 * _D] = h_ref[pl.ds(r1, 1), :]


def _make_extract_call():
    return pl.pallas_call(
        _extract_body,
        out_shape=jax.ShapeDtypeStruct((_B, 2 * _D), jnp.float32),
    )


def kernel(x, edge_index, edge_type, batch, inds, Wrel, Wroot, bias):
    src = edge_index[0].astype(jnp.int32)
    dst = edge_index[1].astype(jnp.int32)
    et = edge_type.reshape(-1).astype(jnp.int32)

    g = dst // _GSZ
    segl = et * _GSZ + (dst - g * _GSZ)
    oh = g[:, None] == jnp.arange(_NG, dtype=jnp.int32)[None, :]
    cnt_g = jnp.sum(oh.astype(jnp.int32), axis=0)
    pc = (cnt_g + _BUCKET_Q - 1) // _BUCKET_Q * _BUCKET_Q
    pb = jnp.concatenate(
        [jnp.zeros((1,), jnp.int32), jnp.cumsum(pc)[:-1].astype(jnp.int32)])
    pad_need_cum = jnp.cumsum(pc - cnt_g)
    padt = _EPB - _E
    padkeys = jnp.searchsorted(
        pad_need_cum, jnp.arange(padt, dtype=jnp.int32), side="right"
    ).astype(jnp.int32)
    keys = jnp.concatenate([g, padkeys])
    vals = jnp.concatenate([(src << 14) | segl,
                            jnp.full((padt,), _LTRASH, jnp.int32)])
    _, vb = lax.sort((keys, vals), num_keys=1, is_stable=True)
    srcb = (vb >> 14).reshape(-1, _GRP)
    segb = (vb & 0x3FFF).reshape(-1, _GRP)
    pbrv = jnp.zeros((16,), jnp.int32).at[:_NG].set(pb // _GRP)
    trv = jnp.zeros((16,), jnp.int32).at[:_NG].set(pc // _BUCKET_Q)

    padc = _EPADC - _E
    etp = jnp.concatenate([et, jnp.zeros((padc,), jnp.int32)])
    etp = (etp + 16 * (jnp.arange(_EPADC, dtype=jnp.int32) % _SPREAD)
           ).reshape(-1, _GRP)
    dstp = jnp.concatenate(
        [dst, jnp.full((padc,), _N, jnp.int32)]).reshape(-1, _GRP)
    eye1 = jnp.zeros((16, _D), jnp.float32).at[
        jnp.arange(16), jnp.arange(16)].set(1.0)
    eye = jnp.tile(eye1, (_SPREAD, 1))
    zsrc = jnp.zeros((_ZB, _D), jnp.float32)
    bias2 = bias.reshape(_L, 1, _D)

    cnt = _make_cnt_call()(eye, etp, dstp, zsrc)

    agg_call = _make_agg_call()
    h = x
    for l in range(_L):
        agg = agg_call(h, srcb, segb, pbrv, trv, zsrc)
        h = _make_layer_call(l < _L - 1)(
            h, agg, cnt, Wroot[l], Wrel[l], bias2[l])

    return _make_extract_call()(h, batch.reshape(_N // 16, 16), inds)
